# Initial kernel scaffold; baseline (speedup 1.0000x reference)
import jax
import jax.numpy as jnp
from jax.experimental import pallas as pl


def kernel(rays_o, rays_d):
    raise NotImplementedError("write your pallas kernel here")


# speedup vs baseline: 1.1743x; 1.1743x over previous
import functools

import jax
import jax.numpy as jnp
from jax import lax
from jax.experimental import pallas as pl
from jax.experimental.pallas import tpu as pltpu
from jax.experimental.pallas import tpu_sc as plsc

N = 2097152
NC = 2
NS = 16
NW = NC * NS
PER_W = N // NW
CHUNK = 4096
N_CHUNKS = PER_W // CHUNK
GROUPS = CHUNK // 16

NEAR = 0.05
FAR = 1000.0
ALT_TOP = 20.0
ALT_BOT = -20.0
_SQ3 = 1.7320508075688772
IRX = 1.0 / (100.0 * _SQ3)
IRZ = 1.0 / (20.0 * _SQ3)


def _collide_body(o_hbm, d_hbm, nears_hbm, fars_hbm, o_v, d_v, n_v, f_v):
    wid = lax.axis_index("s") * NC + lax.axis_index("c")
    w_base = wid * PER_W

    def chunk_body(ci, _):
        base = w_base + ci * CHUNK
        sl = pl.ds(base, CHUNK)
        pltpu.sync_copy(o_hbm.at[:, sl], o_v)
        pltpu.sync_copy(d_hbm.at[:, sl], d_v)

        def grp(g, _):
            s = g * 16
            gs = pl.ds(s, 16)
            ox = o_v[0, gs]
            oy = o_v[1, gs]
            oz = o_v[2, gs]
            dx = d_v[0, gs]
            dy = d_v[1, gs]
            dz = d_v[2, gs]

            goes_down = dz < 0.0
            b1 = jnp.logical_and(oz > ALT_TOP, goes_down)
            b2 = jnp.logical_and(oz > ALT_BOT, goes_down)
            den1 = jnp.where(b1, dz, 1.0)
            den2 = jnp.where(b2, dz, 1.0)
            t1 = (ALT_TOP - oz) / den1
            t2 = (ALT_BOT - oz) / den2
            nears = jnp.where(b1, t1, NEAR)
            fars = jnp.where(b2, t2, FAR)

            osx = ox * IRX
            osy = oy * IRX
            osz = oz * IRZ
            dsx = dx * IRX
            dsy = dy * IRX
            dsz = dz * IRZ
            a = dsx * dsx + dsy * dsy + dsz * dsz
            b = 2.0 * (osx * dsx + osy * dsy + osz * dsz)
            c = osx * osx + osy * osy + osz * osz - 1.0
            disc = b * b - 4.0 * a * c
            mask = disc > 0.0
            dsafe = jnp.where(mask, disc, 1.0)
            bits = lax.bitcast_convert_type(dsafe, jnp.int32)
            bits = jnp.int32(0x5F3759DF) - lax.shift_right_logical(bits, 1)
            y = lax.bitcast_convert_type(bits, jnp.float32)
            hx = 0.5 * dsafe
            y = y * (1.5 - hx * y * y)
            y = y * (1.5 - hx * y * y)
            y = y * (1.5 - hx * y * y)
            sq = jnp.where(mask, dsafe * y, 0.0)
            sphere_far = (-b + sq) / (2.0 * a)
            sphere_far = jnp.where(mask, sphere_far, 0.0)

            fars2 = jnp.minimum(fars, sphere_far)
            nears2 = jnp.maximum(nears, NEAR)
            fars3 = jnp.minimum(jnp.maximum(fars2, nears2 + 1e-06), FAR)

            n_v[gs] = nears2
            f_v[gs] = fars3
            return 0

        lax.fori_loop(0, GROUPS, grp, 0)
        pltpu.sync_copy(n_v, nears_hbm.at[sl])
        pltpu.sync_copy(f_v, fars_hbm.at[sl])
        return 0

    lax.fori_loop(0, N_CHUNKS, chunk_body, 0)


@jax.jit
def _collide(o_t, d_t):
    mesh = plsc.VectorSubcoreMesh(core_axis_name="c", subcore_axis_name="s")
    fn = functools.partial(
        pl.kernel,
        mesh=mesh,
        out_type=(
            jax.ShapeDtypeStruct((N,), jnp.float32),
            jax.ShapeDtypeStruct((N,), jnp.float32),
        ),
        scratch_types=[
            pltpu.VMEM((3, CHUNK), jnp.float32),
            pltpu.VMEM((3, CHUNK), jnp.float32),
            pltpu.VMEM((CHUNK,), jnp.float32),
            pltpu.VMEM((CHUNK,), jnp.float32),
        ],
    )(_collide_body)
    return fn(o_t, d_t)


def kernel(rays_o, rays_d):
    nears, fars = _collide(rays_o.T, rays_d.T)
    return nears.reshape(N, 1), fars.reshape(N, 1)


# speedup vs baseline: 1.2354x; 1.0520x over previous
import functools

import jax
import jax.numpy as jnp
from jax import lax
from jax.experimental import pallas as pl
from jax.experimental.pallas import tpu as pltpu
from jax.experimental.pallas import tpu_sc as plsc

N = 2097152
NC = 2
NS = 16
NW = NC * NS
PER_W = N // NW
CHUNK = 4096
N_CHUNKS = PER_W // CHUNK
GROUPS = CHUNK // 16

NEAR = 0.05
FAR = 1000.0
ALT_TOP = 20.0
ALT_BOT = -20.0
_SQ3 = 1.7320508075688772
IRX = 1.0 / (100.0 * _SQ3)
IRZ = 1.0 / (20.0 * _SQ3)
KXY = IRX * IRX
KZ = IRZ * IRZ


def _collide_body(o_hbm, d_hbm, nears_hbm, fars_hbm, o_v, d_v, n_v, f_v):
    wid = lax.axis_index("s") * NC + lax.axis_index("c")
    w_base = wid * PER_W

    def chunk_body(ci, _):
        base = w_base + ci * CHUNK
        sl = pl.ds(base, CHUNK)
        pltpu.sync_copy(o_hbm.at[:, sl], o_v)
        pltpu.sync_copy(d_hbm.at[:, sl], d_v)

        def grp(g, _):
            s = g * 16
            gs = pl.ds(s, 16)
            ox = o_v[0, gs]
            oy = o_v[1, gs]
            oz = o_v[2, gs]
            dx = d_v[0, gs]
            dy = d_v[1, gs]
            dz = d_v[2, gs]

            rdz = 1.0 / dz
            goes_down = dz < 0.0
            b1 = jnp.logical_and(oz > ALT_TOP, goes_down)
            b2 = jnp.logical_and(oz > ALT_BOT, goes_down)
            t1 = (ALT_TOP - oz) * rdz
            t2 = (ALT_BOT - oz) * rdz
            nears = jnp.where(b1, t1, NEAR)
            fars = jnp.where(b2, t2, FAR)

            a = KXY * (dx * dx + dy * dy) + KZ * (dz * dz)
            hb = KXY * (ox * dx + oy * dy) + KZ * (oz * dz)
            c = KXY * (ox * ox + oy * oy) + KZ * (oz * oz) - 1.0
            disc = hb * hb - a * c
            mask = disc > 0.0
            dsafe = jnp.where(mask, disc, 1.0)
            bits = lax.bitcast_convert_type(dsafe, jnp.int32)
            bits = jnp.int32(0x5F3759DF) - lax.shift_right_logical(bits, 1)
            y = lax.bitcast_convert_type(bits, jnp.float32)
            hx = 0.5 * dsafe
            y = y * (1.5 - hx * y * y)
            y = y * (1.5 - hx * y * y)
            y = y * (1.5 - hx * y * y)
            sq = dsafe * y
            sphere_far = jnp.where(mask, (sq - hb) * (1.0 / a), 0.0)

            fars2 = jnp.minimum(fars, sphere_far)
            nears2 = jnp.maximum(nears, NEAR)
            fars3 = jnp.minimum(jnp.maximum(fars2, nears2 + 1e-06), FAR)

            n_v[gs] = nears2
            f_v[gs] = fars3
            return 0

        lax.fori_loop(0, GROUPS, grp, 0, unroll=8)
        pltpu.sync_copy(n_v, nears_hbm.at[sl])
        pltpu.sync_copy(f_v, fars_hbm.at[sl])
        return 0

    lax.fori_loop(0, N_CHUNKS, chunk_body, 0)


@jax.jit
def _collide(o_t, d_t):
    mesh = plsc.VectorSubcoreMesh(core_axis_name="c", subcore_axis_name="s")
    fn = functools.partial(
        pl.kernel,
        mesh=mesh,
        out_type=(
            jax.ShapeDtypeStruct((N,), jnp.float32),
            jax.ShapeDtypeStruct((N,), jnp.float32),
        ),
        scratch_types=[
            pltpu.VMEM((3, CHUNK), jnp.float32),
            pltpu.VMEM((3, CHUNK), jnp.float32),
            pltpu.VMEM((CHUNK,), jnp.float32),
            pltpu.VMEM((CHUNK,), jnp.float32),
        ],
    )(_collide_body)
    return fn(o_t, d_t)


def kernel(rays_o, rays_d):
    nears, fars = _collide(rays_o.T, rays_d.T)
    return nears.reshape(N, 1), fars.reshape(N, 1)


# speedup vs baseline: 1.5244x; 1.2339x over previous
import functools

import jax
import jax.numpy as jnp
from jax import lax
from jax.experimental import pallas as pl
from jax.experimental.pallas import tpu as pltpu
from jax.experimental.pallas import tpu_sc as plsc

N = 2097152
NC = 2
NS = 16
NW = NC * NS
PER_W = N // NW
CHUNK = 4096
N_CHUNKS = PER_W // CHUNK
GROUPS = CHUNK // 16

NEAR = 0.05
FAR = 1000.0
ALT_TOP = 20.0
ALT_BOT = -20.0
_SQ3 = 1.7320508075688772
IRX = 1.0 / (100.0 * _SQ3)
IRZ = 1.0 / (20.0 * _SQ3)
KXY = IRX * IRX
KZ = IRZ * IRZ


def _collide_body(o_hbm, d_hbm, nears_hbm, fars_hbm, o_v, d_v, n_v, f_v,
                  sio0, sio1, sid0, sid1, son0, son1, sof0, sof1):
    wid = lax.axis_index("s") * NC + lax.axis_index("c")
    w_base = wid * PER_W
    sio = (sio0, sio1)
    sid = (sid0, sid1)
    son = (son0, son1)
    sof = (sof0, sof1)

    def in_copies(ci, b):
        sl = pl.ds(w_base + ci * CHUNK, CHUNK)
        return (
            pltpu.make_async_copy(o_hbm.at[:, sl], o_v.at[b], sio[b]),
            pltpu.make_async_copy(d_hbm.at[:, sl], d_v.at[b], sid[b]),
        )

    def out_copies(ci, b):
        sl = pl.ds(w_base + ci * CHUNK, CHUNK)
        return (
            pltpu.make_async_copy(n_v.at[b], nears_hbm.at[sl], son[b]),
            pltpu.make_async_copy(f_v.at[b], fars_hbm.at[sl], sof[b]),
        )

    for cp in in_copies(0, 0):
        cp.start()

    def chunk_pair(ci2, _):
        for b in (0, 1):
            ci = ci2 * 2 + b
            nxt = ci + 1

            @pl.when(nxt < N_CHUNKS)
            def _():
                for cp in in_copies(nxt, 1 - b):
                    cp.start()

            for cp in in_copies(ci, b):
                cp.wait()

            @pl.when(ci >= 2)
            def _():
                for cp in out_copies(ci, b):
                    cp.wait()

            _compute_chunk(o_v, d_v, n_v, f_v, b)

            for cp in out_copies(ci, b):
                cp.start()
        return 0

    lax.fori_loop(0, N_CHUNKS // 2, chunk_pair, 0)
    for b in (0, 1):
        for cp in out_copies(N_CHUNKS - 2 + b, b):
            cp.wait()


def _compute_chunk(o_v, d_v, n_v, f_v, b):
        def grp(g, _):
            s = g * 16
            gs = pl.ds(s, 16)
            ox = o_v[b, 0, gs]
            oy = o_v[b, 1, gs]
            oz = o_v[b, 2, gs]
            dx = d_v[b, 0, gs]
            dy = d_v[b, 1, gs]
            dz = d_v[b, 2, gs]

            rdz = 1.0 / dz
            goes_down = dz < 0.0
            b1 = jnp.logical_and(oz > ALT_TOP, goes_down)
            b2 = jnp.logical_and(oz > ALT_BOT, goes_down)
            t1 = (ALT_TOP - oz) * rdz
            t2 = (ALT_BOT - oz) * rdz
            nears = jnp.where(b1, t1, NEAR)
            fars = jnp.where(b2, t2, FAR)

            a = KXY * (dx * dx + dy * dy) + KZ * (dz * dz)
            hb = KXY * (ox * dx + oy * dy) + KZ * (oz * dz)
            c = KXY * (ox * ox + oy * oy) + KZ * (oz * oz) - 1.0
            disc = hb * hb - a * c
            mask = disc > 0.0
            dsafe = jnp.where(mask, disc, 1.0)
            bits = lax.bitcast_convert_type(dsafe, jnp.int32)
            bits = jnp.int32(0x5F3759DF) - lax.shift_right_logical(bits, 1)
            y = lax.bitcast_convert_type(bits, jnp.float32)
            hx = 0.5 * dsafe
            y = y * (1.5 - hx * y * y)
            y = y * (1.5 - hx * y * y)
            y = y * (1.5 - hx * y * y)
            sq = dsafe * y
            sphere_far = jnp.where(mask, (sq - hb) * (1.0 / a), 0.0)

            fars2 = jnp.minimum(fars, sphere_far)
            nears2 = jnp.maximum(nears, NEAR)
            fars3 = jnp.minimum(jnp.maximum(fars2, nears2 + 1e-06), FAR)

            n_v[b, gs] = nears2
            f_v[b, gs] = fars3
            return 0

        lax.fori_loop(0, GROUPS, grp, 0, unroll=8)


@jax.jit
def _collide(o_t, d_t):
    mesh = plsc.VectorSubcoreMesh(core_axis_name="c", subcore_axis_name="s")
    fn = functools.partial(
        pl.kernel,
        mesh=mesh,
        out_type=(
            jax.ShapeDtypeStruct((N,), jnp.float32),
            jax.ShapeDtypeStruct((N,), jnp.float32),
        ),
        scratch_types=[
            pltpu.VMEM((2, 3, CHUNK), jnp.float32),
            pltpu.VMEM((2, 3, CHUNK), jnp.float32),
            pltpu.VMEM((2, CHUNK), jnp.float32),
            pltpu.VMEM((2, CHUNK), jnp.float32),
        ] + [pltpu.SemaphoreType.DMA] * 8,
    )(_collide_body)
    return fn(o_t, d_t)


def kernel(rays_o, rays_d):
    nears, fars = _collide(rays_o.T, rays_d.T)
    return nears.reshape(N, 1), fars.reshape(N, 1)
